# baseline (speedup 1.0000x reference)
import functools

import jax
import jax.numpy as jnp
from jax import lax
from jax.experimental import pallas as pl
from jax.experimental.pallas import tpu as pltpu
from jax.experimental.pallas import tpu_sc as plsc

B, N, D, K = 4, 2048, 256, 16
XL = 128
TW = D + XL
MA = 512
MB = 512
MC = 128
NC, NS = 2, 16
NW = NC * NS
CH = 128


def _bits(x):
    return lax.bitcast_convert_type(x, jnp.uint32)


def _proj_body(f_ref, x128_ref, w1_ref, b1_ref, wq_ref, wk_ref, wv_ref,
               q_ref, t_ref):
    dot = functools.partial(jnp.dot, preferred_element_type=jnp.float32)
    x = dot(f_ref[...], w1_ref[...]) + b1_ref[...]
    q_ref[...] = dot(x, wq_ref[...])
    kb = _bits(dot(x, wk_ref[...]).astype(jnp.bfloat16).astype(jnp.float32))
    vb = _bits(dot(x, wv_ref[...]).astype(jnp.bfloat16).astype(jnp.float32))
    t_ref[:, :D] = lax.bitcast_convert_type(kb | (vb >> 16), jnp.int32)
    t_ref[:, D:] = lax.bitcast_convert_type(x128_ref[...], jnp.int32)


def _proj(features, xyz128, fc1_w, fc1_b, wq, wk, wv):
    rows = B * N
    wspec = pl.BlockSpec((D, D), lambda i: (0, 0))
    return pl.pallas_call(
        _proj_body,
        grid=(rows // MA,),
        in_specs=[
            pl.BlockSpec((MA, D), lambda i: (i, 0)),
            pl.BlockSpec((MA, XL), lambda i: (i, 0)),
            wspec,
            pl.BlockSpec((1, D), lambda i: (0, 0)),
            wspec, wspec, wspec,
        ],
        out_specs=[
            pl.BlockSpec((MA, D), lambda i: (i, 0)),
            pl.BlockSpec((MA, TW), lambda i: (i, 0)),
        ],
        out_shape=[
            jax.ShapeDtypeStruct((rows, D), jnp.float32),
            jax.ShapeDtypeStruct((rows, TW), jnp.int32),
        ],
    )(features.reshape(rows, D), xyz128, fc1_w, fc1_b.reshape(1, D),
      wq, wk, wv)


def _knn_body(xq_ref, xt_ref, idx_ref, *, base):
    xq = xq_ref[0]
    xt = xt_ref[0]
    sqq = jnp.sum(xq * xq, axis=1, keepdims=True)
    sqf = jnp.sum(xt * xt, axis=0, keepdims=True)
    dots = lax.dot_general(xq, xt, (((1,), (0,)), ((), ())),
                           preferred_element_type=jnp.float32)
    d = dots * (-2.0) + sqq + sqf
    iota = lax.broadcasted_iota(jnp.int32, d.shape, 1).astype(jnp.float32)
    io = jnp.concatenate(
        [lax.broadcasted_iota(jnp.int32, (N, 1), 0).astype(jnp.float32),
         jnp.ones((N, 1), jnp.float32)], axis=1)
    inf = jnp.float32(jnp.inf)
    big = jnp.float32(N)
    for j in range(K):
        m = jnp.min(d, axis=1, keepdims=True)
        eq = d == m
        r = lax.dot_general(eq.astype(jnp.float32), io,
                            (((1,), (0,)), ((), ())),
                            preferred_element_type=jnp.float32)
        idx_ref[:, j:j + 1] = r[:, 0:1].astype(jnp.int32)

        @pl.when(jnp.max(r[:, 1]) > 1.5)
        def _():
            slow = jnp.min(jnp.where(eq, iota, big), axis=1, keepdims=True)
            idx_ref[:, j:j + 1] = slow.astype(jnp.int32)

        if j + 1 < K:
            ij = idx_ref[:, j:j + 1].astype(jnp.float32)
            d = jnp.where(iota == ij, inf, d)
    idx_ref[...] = idx_ref[...] + base


def _knn1(xyz, xyz_t, b):
    return pl.pallas_call(
        functools.partial(_knn_body, base=b * N),
        grid=(N // MB,),
        in_specs=[
            pl.BlockSpec((1, MB, 3), lambda i: (b, i, 0)),
            pl.BlockSpec((1, 3, N), lambda i: (b, 0, 0)),
        ],
        out_specs=pl.BlockSpec((MB, K), lambda i: (i, 0)),
        out_shape=jax.ShapeDtypeStruct((N, K), jnp.int32),
    )(xyz, xyz_t)


def _sc_gather_body(tab_ref, idx_ref, out_ref, idx_v, buf, sem):
    wid = lax.axis_index("s") * NC + lax.axis_index("c")
    per_w = (N * K) // NW

    def chunk(ci, carry):
        off = wid * per_w + ci * CH
        pltpu.sync_copy(idx_ref.at[pl.ds(off, CH)], idx_v)
        pltpu.async_copy(tab_ref.at[idx_v], buf, sem).wait()
        pltpu.sync_copy(buf, out_ref.at[pl.ds(off, CH)])
        return carry

    lax.fori_loop(0, per_w // CH, chunk, 0)


def _sc_gather(table, idx_flat):
    tot = N * K
    f = pl.kernel(
        _sc_gather_body,
        mesh=plsc.VectorSubcoreMesh(core_axis_name="c", subcore_axis_name="s"),
        out_type=jax.ShapeDtypeStruct((tot, TW), jnp.int32),
        scratch_types=[
            pltpu.VMEM((CH,), jnp.int32),
            pltpu.VMEM((CH, TW), jnp.int32),
            pltpu.SemaphoreType.DMA,
        ],
    )
    return f(table, idx_flat)


def _attn_body(q_ref, tg_ref, xq_ref, dw1_ref, db1_ref, dw2_ref,
               db2_ref, g1_ref, gb1_ref, g2_ref, gb2_ref, f2_ref, f2b_ref,
               out_ref):
    dot = functools.partial(jnp.dot, preferred_element_type=jnp.float32)

    rows = MC * K
    w = _bits(tg_ref[:, :D])
    kf = lax.bitcast_convert_type(w & jnp.uint32(0xFFFF0000), jnp.float32)
    v = lax.bitcast_convert_type(w << 16, jnp.float32)
    xg = lax.bitcast_convert_type(tg_ref[:, D:], jnp.float32)
    delta = (xq_ref[...][:, None, :] - xg.reshape(MC, K, XL))
    h1 = jax.nn.relu(dot(delta.reshape(rows, XL), dw1_ref[...]) + db1_ref[...])
    pos = dot(h1, dw2_ref[...]) + db2_ref[...]
    q3 = q_ref[...][:, None, :]
    t3 = q3 - kf.reshape(MC, K, D) + pos.reshape(MC, K, D)
    a1 = jax.nn.relu(dot(t3.reshape(rows, D), g1_ref[...]) + gb1_ref[...])
    a3 = (dot(a1, g2_ref[...]) + gb2_ref[...]).reshape(MC, K, D)
    e = jnp.exp(a3)
    attn = e / jnp.sum(e, axis=1, keepdims=True)
    agg = jnp.sum(attn * (v.reshape(MC, K, D) + pos.reshape(MC, K, D)), axis=1)
    out_ref[...] = dot(agg, f2_ref[...]) + f2b_ref[...]


def _attn(q_flat, tab_g, xyz128, dw1p, db1, dw2, db2, g1, gb1, g2, gb2,
          fc2_w, fc2_b, b):
    rows = N
    off = b * (N // MC)
    wspec = pl.BlockSpec((D, D), lambda i: (0, 0))
    bspec = pl.BlockSpec((1, D), lambda i: (0, 0))
    out = pl.pallas_call(
        _attn_body,
        grid=(rows // MC,),
        in_specs=[
            pl.BlockSpec((MC, D), lambda i: (off + i, 0)),
            pl.BlockSpec((MC * K, TW), lambda i: (i, 0)),
            pl.BlockSpec((MC, XL), lambda i: (off + i, 0)),
            pl.BlockSpec((XL, D), lambda i: (0, 0)), bspec,
            wspec, bspec,
            wspec, bspec,
            wspec, bspec,
            wspec, bspec,
        ],
        out_specs=pl.BlockSpec((MC, D), lambda i: (i, 0)),
        out_shape=jax.ShapeDtypeStruct((rows, D), jnp.float32),
    )(q_flat, tab_g, xyz128, dw1p, db1.reshape(1, D), dw2,
      db2.reshape(1, D), g1, gb1.reshape(1, D), g2, gb2.reshape(1, D),
      fc2_w, fc2_b.reshape(1, D))
    return out


def kernel(xyz, features, fc1_w, fc1_b, fc2_w, fc2_b, delta_w1, delta_b1,
           delta_w2, delta_b2, gamma_w1, gamma_b1, gamma_w2, gamma_b2,
           wq, wk, wv):
    xyz128 = jnp.pad(xyz.reshape(B * N, 3), ((0, 0), (0, XL - 3)))
    q_flat, table = _proj(features, xyz128, fc1_w, fc1_b, wq, wk, wv)
    dw1p = jnp.pad(delta_w1, ((0, XL - 3), (0, 0)))
    xyz_t = jnp.swapaxes(xyz, 1, 2)
    outs = []
    for b in range(B):
        idx_b = _knn1(xyz, xyz_t, b)
        tab_g = _sc_gather(table, idx_b.reshape(N * K))
        outs.append(_attn(q_flat, tab_g, xyz128, dw1p, delta_b1,
                          delta_w2, delta_b2, gamma_w1, gamma_b1,
                          gamma_w2 * (1.0 / 16.0), gamma_b2 * (1.0 / 16.0),
                          fc2_w, fc2_b, b))
    return jnp.stack(outs)


# speedup vs baseline: 1.8394x; 1.8394x over previous
import functools

import jax
import jax.numpy as jnp
from jax import lax
from jax.experimental import pallas as pl
from jax.experimental.pallas import tpu as pltpu
from jax.experimental.pallas import tpu_sc as plsc

B, N, D, K = 4, 2048, 256, 16
XL = 128
TW = D + XL
MA = 512
MB = 512
MC = 128
NC, NS = 2, 16
NW = NC * NS
CH = 128


def _bits(x):
    return lax.bitcast_convert_type(x, jnp.uint32)


def _proj_body(f_ref, x128_ref, w1_ref, b1_ref, wq_ref, wk_ref, wv_ref,
               q_ref, t_ref):
    dot = functools.partial(jnp.dot, preferred_element_type=jnp.float32)
    x = dot(f_ref[...], w1_ref[...]) + b1_ref[...]
    q_ref[...] = dot(x, wq_ref[...])
    kb = _bits(dot(x, wk_ref[...]).astype(jnp.bfloat16).astype(jnp.float32))
    vb = _bits(dot(x, wv_ref[...]).astype(jnp.bfloat16).astype(jnp.float32))
    t_ref[:, :D] = lax.bitcast_convert_type(kb | (vb >> 16), jnp.int32)
    t_ref[:, D:] = lax.bitcast_convert_type(x128_ref[...], jnp.int32)


def _proj(features, xyz128, fc1_w, fc1_b, wq, wk, wv):
    rows = B * N
    wspec = pl.BlockSpec((D, D), lambda i: (0, 0))
    return pl.pallas_call(
        _proj_body,
        grid=(rows // MA,),
        in_specs=[
            pl.BlockSpec((MA, D), lambda i: (i, 0)),
            pl.BlockSpec((MA, XL), lambda i: (i, 0)),
            wspec,
            pl.BlockSpec((1, D), lambda i: (0, 0)),
            wspec, wspec, wspec,
        ],
        out_specs=[
            pl.BlockSpec((MA, D), lambda i: (i, 0)),
            pl.BlockSpec((MA, TW), lambda i: (i, 0)),
        ],
        out_shape=[
            jax.ShapeDtypeStruct((rows, D), jnp.float32),
            jax.ShapeDtypeStruct((rows, TW), jnp.int32),
        ],
    )(features.reshape(rows, D), xyz128, fc1_w, fc1_b.reshape(1, D),
      wq, wk, wv)


def _knn_body(xq_ref, xt_ref, idx_ref, *, base):
    xq = xq_ref[0]
    xt = xt_ref[0]
    sqq = jnp.sum(xq * xq, axis=1, keepdims=True)
    sqf = jnp.sum(xt * xt, axis=0, keepdims=True)
    dots = lax.dot_general(xq, xt, (((1,), (0,)), ((), ())),
                           preferred_element_type=jnp.float32)
    d = dots * (-2.0) + sqq + sqf
    iota = lax.broadcasted_iota(jnp.int32, d.shape, 1).astype(jnp.float32)
    inf = jnp.float32(jnp.inf)
    big = jnp.float32(N)
    idx_cols = []
    for j in range(K):
        m = jnp.min(d, axis=1, keepdims=True)
        cand = jnp.where(d == m, iota, big)
        ij = jnp.min(cand, axis=1, keepdims=True)
        if j + 1 < K:
            d = jnp.where(iota == ij, inf, d)
        idx_cols.append(ij)
    idx = jnp.concatenate(idx_cols, axis=1).astype(jnp.int32) + base
    idx_ref[...] = idx


def _knn1(xyz, xyz_t, b):
    return pl.pallas_call(
        functools.partial(_knn_body, base=b * N),
        grid=(N // MB,),
        in_specs=[
            pl.BlockSpec((1, MB, 3), lambda i: (b, i, 0)),
            pl.BlockSpec((1, 3, N), lambda i: (b, 0, 0)),
        ],
        out_specs=pl.BlockSpec((MB, K), lambda i: (i, 0)),
        out_shape=jax.ShapeDtypeStruct((N, K), jnp.int32),
    )(xyz, xyz_t)


def _sc_gather_body(tab_ref, idx_ref, out_ref, idx_v, buf, sem):
    wid = lax.axis_index("s") * NC + lax.axis_index("c")
    per_w = (N * K) // NW

    def chunk(ci, carry):
        off = wid * per_w + ci * CH
        pltpu.sync_copy(idx_ref.at[pl.ds(off, CH)], idx_v)
        pltpu.async_copy(tab_ref.at[idx_v], buf, sem).wait()
        pltpu.sync_copy(buf, out_ref.at[pl.ds(off, CH)])
        return carry

    lax.fori_loop(0, per_w // CH, chunk, 0)


def _sc_gather(table, idx_flat):
    tot = N * K
    f = pl.kernel(
        _sc_gather_body,
        mesh=plsc.VectorSubcoreMesh(core_axis_name="c", subcore_axis_name="s"),
        out_type=jax.ShapeDtypeStruct((tot, TW), jnp.int32),
        scratch_types=[
            pltpu.VMEM((CH,), jnp.int32),
            pltpu.VMEM((CH, TW), jnp.int32),
            pltpu.SemaphoreType.DMA,
        ],
    )
    return f(table, idx_flat)


def _attn_body(q_ref, tg_ref, xq_ref, dw1_ref, db1_ref, dw2_ref,
               db2_ref, g1_ref, gb1_ref, g2_ref, gb2_ref, f2_ref, f2b_ref,
               out_ref):
    dot = functools.partial(jnp.dot, preferred_element_type=jnp.float32)

    rows = MC * K
    w = _bits(tg_ref[:, :D])
    kf = lax.bitcast_convert_type(w & jnp.uint32(0xFFFF0000), jnp.float32)
    v = lax.bitcast_convert_type(w << 16, jnp.float32)
    xg = lax.bitcast_convert_type(tg_ref[:, D:], jnp.float32)
    delta = (xq_ref[...][:, None, :] - xg.reshape(MC, K, XL))
    h1 = jax.nn.relu(dot(delta.reshape(rows, XL), dw1_ref[...]) + db1_ref[...])
    pos = dot(h1, dw2_ref[...]) + db2_ref[...]
    q3 = q_ref[...][:, None, :]
    t3 = q3 - kf.reshape(MC, K, D) + pos.reshape(MC, K, D)
    a1 = jax.nn.relu(dot(t3.reshape(rows, D), g1_ref[...]) + gb1_ref[...])
    a3 = (dot(a1, g2_ref[...]) + gb2_ref[...]).reshape(MC, K, D)
    e = jnp.exp(a3)
    attn = e / jnp.sum(e, axis=1, keepdims=True)
    agg = jnp.sum(attn * (v.reshape(MC, K, D) + pos.reshape(MC, K, D)), axis=1)
    out_ref[...] = dot(agg, f2_ref[...]) + f2b_ref[...]


def _attn(q_flat, tab_g, xyz128, dw1p, db1, dw2, db2, g1, gb1, g2, gb2,
          fc2_w, fc2_b, b):
    rows = N
    off = b * (N // MC)
    wspec = pl.BlockSpec((D, D), lambda i: (0, 0))
    bspec = pl.BlockSpec((1, D), lambda i: (0, 0))
    out = pl.pallas_call(
        _attn_body,
        grid=(rows // MC,),
        in_specs=[
            pl.BlockSpec((MC, D), lambda i: (off + i, 0)),
            pl.BlockSpec((MC * K, TW), lambda i: (i, 0)),
            pl.BlockSpec((MC, XL), lambda i: (off + i, 0)),
            pl.BlockSpec((XL, D), lambda i: (0, 0)), bspec,
            wspec, bspec,
            wspec, bspec,
            wspec, bspec,
            wspec, bspec,
        ],
        out_specs=pl.BlockSpec((MC, D), lambda i: (i, 0)),
        out_shape=jax.ShapeDtypeStruct((rows, D), jnp.float32),
    )(q_flat, tab_g, xyz128, dw1p, db1.reshape(1, D), dw2,
      db2.reshape(1, D), g1, gb1.reshape(1, D), g2, gb2.reshape(1, D),
      fc2_w, fc2_b.reshape(1, D))
    return out


def kernel(xyz, features, fc1_w, fc1_b, fc2_w, fc2_b, delta_w1, delta_b1,
           delta_w2, delta_b2, gamma_w1, gamma_b1, gamma_w2, gamma_b2,
           wq, wk, wv):
    xyz128 = jnp.pad(xyz.reshape(B * N, 3), ((0, 0), (0, XL - 3)))
    q_flat, table = _proj(features, xyz128, fc1_w, fc1_b, wq, wk, wv)
    dw1p = jnp.pad(delta_w1, ((0, XL - 3), (0, 0)))
    xyz_t = jnp.swapaxes(xyz, 1, 2)
    outs = []
    for b in range(B):
        idx_b = _knn1(xyz, xyz_t, b)
        tab_g = _sc_gather(table, idx_b.reshape(N * K))
        outs.append(_attn(q_flat, tab_g, xyz128, dw1p, delta_b1,
                          delta_w2, delta_b2, gamma_w1, gamma_b1,
                          gamma_w2 * (1.0 / 16.0), gamma_b2 * (1.0 / 16.0),
                          fc2_w, fc2_b, b))
    return jnp.stack(outs)
